# baseline (speedup 1.0000x reference)
import jax
import jax.numpy as jnp
from jax.experimental import pallas as pl
from jax.experimental.pallas import tpu as pltpu

_THRESHOLD = 3e-05
_COEF = 0.01
_CHANNELS = 64.0
_PROWS = 4096
_NCHUNK = 4
_CHUNK = _PROWS // _NCHUNK


def _sparse_max_loss_kernel(xp_hbm, xh_ref, o_ref, buf, sem):
    copies = []
    for i in range(_NCHUNK):
        rows = pl.ds(i * _CHUNK, _CHUNK)
        c = pltpu.make_async_copy(xp_hbm.at[rows, :], buf.at[rows, :], sem.at[i])
        c.start()
        copies.append(c)

    tn = (((0,), (0,)), ((), ()))
    ap = jnp.zeros((128, 128), jnp.float32)
    bp = jnp.zeros((128, 128), jnp.float32)
    mp = jnp.zeros((1, 128), jnp.float32)
    for i in range(_NCHUNK):
        copies[i].wait()
        x = buf[pl.ds(i * _CHUNK, _CHUNK), :]
        condf = (x > _THRESHOLD).astype(jnp.float32)
        condb = condf.astype(jnp.bfloat16)
        xb = x.astype(jnp.bfloat16)
        xsqb = (x * x).astype(jnp.bfloat16)
        ap += jax.lax.dot_general(condb, xb, tn, preferred_element_type=jnp.float32)
        bp += jax.lax.dot_general(condb, xsqb, tn, preferred_element_type=jnp.float32)
        mp += jnp.sum(condf, axis=0, keepdims=True)

    a = ap[:64, :64] + ap[64:, 64:]
    b = bp[:64, :64] + bp[64:, 64:]
    m = mp[:, :64] + mp[:, 64:]

    xh = xh_ref[...]
    xht = xh.T
    s_h = jnp.sum(xht, axis=0, keepdims=True)
    q_h = jnp.sum(xht * xht, axis=0, keepdims=True)

    total = (
        _CHANNELS * jnp.sum(m, keepdims=True)
        - (jnp.sum(a, keepdims=True) + jnp.sum(m * s_h, keepdims=True))
        * (1.0 / 32.0)
        + (jnp.sum(b, keepdims=True) + jnp.sum(m * q_h, keepdims=True))
        * (1.0 / 4096.0)
        + jnp.sum(a * xh, keepdims=True) * (1.0 / 2048.0)
    )
    o_ref[...] = (_COEF / _CHANNELS) * jnp.sqrt(total)


def kernel(x):
    xp = jnp.reshape(x, (_PROWS, 128))
    out = pl.pallas_call(
        _sparse_max_loss_kernel,
        grid=(1,),
        in_specs=[
            pl.BlockSpec(memory_space=pltpu.MemorySpace.HBM),
            pl.BlockSpec((64, 64), lambda i: (0, 0)),
        ],
        out_shape=jax.ShapeDtypeStruct((1, 1), jnp.float32),
        out_specs=pl.BlockSpec((1, 1), lambda i: (0, 0)),
        scratch_shapes=[
            pltpu.VMEM((_PROWS, 128), jnp.float32),
            pltpu.SemaphoreType.DMA((_NCHUNK,)),
        ],
    )(xp, x)
    return jnp.reshape(out, ())


# speedup vs baseline: 1.2091x; 1.2091x over previous
import jax
import jax.numpy as jnp
from jax.experimental import pallas as pl
from jax.experimental.pallas import tpu as pltpu

_THRESHOLD = 3e-05
_COEF = 0.01
_CHANNELS = 64.0
_ROWS = 8192
_CHUNKS = (1024, 2560, 2560, 1536, 512)
_NCHUNK = len(_CHUNKS)
_STARTS = tuple(sum(_CHUNKS[:i]) for i in range(_NCHUNK))


def _sparse_max_loss_kernel(x_hbm, o_ref, buf, sem):
    copies = []
    for i in range(_NCHUNK):
        rows = pl.ds(_STARTS[i], _CHUNKS[i])
        c = pltpu.make_async_copy(x_hbm.at[rows, :], buf.at[rows, :], sem.at[i])
        c.start()
        copies.append(c)

    tn = (((0,), (0,)), ((), ()))
    a = jnp.zeros((64, 64), jnp.float32)
    b = jnp.zeros((64, 64), jnp.float32)
    m = jnp.zeros((1, 64), jnp.float32)
    xh = None
    for i in range(_NCHUNK):
        copies[i].wait()
        x = buf[pl.ds(_STARTS[i], _CHUNKS[i]), :]
        if i == 0:
            xh = x[:64, :]
        condf = (x > _THRESHOLD).astype(jnp.float32)
        condb = condf.astype(jnp.bfloat16)
        xb = x.astype(jnp.bfloat16)
        xsqb = (x * x).astype(jnp.bfloat16)
        a += jax.lax.dot_general(condb, xb, tn, preferred_element_type=jnp.float32)
        b += jax.lax.dot_general(condb, xsqb, tn, preferred_element_type=jnp.float32)
        m += jnp.sum(condf, axis=0, keepdims=True)

    xht = xh.T
    s_h = jnp.sum(xht, axis=0, keepdims=True)
    q_h = jnp.sum(xht * xht, axis=0, keepdims=True)

    total = (
        _CHANNELS * jnp.sum(m, keepdims=True)
        - (jnp.sum(a, keepdims=True) + jnp.sum(m * s_h, keepdims=True))
        * (1.0 / 32.0)
        + (jnp.sum(b, keepdims=True) + jnp.sum(m * q_h, keepdims=True))
        * (1.0 / 4096.0)
        + jnp.sum(a * xh, keepdims=True) * (1.0 / 2048.0)
    )
    o_ref[...] = (_COEF / _CHANNELS) * jnp.sqrt(total)


def kernel(x):
    out = pl.pallas_call(
        _sparse_max_loss_kernel,
        in_specs=[pl.BlockSpec(memory_space=pltpu.MemorySpace.HBM)],
        out_shape=jax.ShapeDtypeStruct((1, 1), jnp.float32),
        scratch_shapes=[
            pltpu.VMEM((_ROWS, 64), jnp.float32),
            pltpu.SemaphoreType.DMA((_NCHUNK,)),
        ],
    )(x)
    return jnp.reshape(out, ())
